# baseline (speedup 1.0000x reference)
import jax
import jax.numpy as jnp
from jax.experimental import pallas as pl
from jax.experimental.pallas import tpu as pltpu

D_MODEL = 4096
N_HEADS = 64
TOP_K = 8
BLK = 1024
DCHUNK = 1024
ND = D_MODEL // DCHUNK


def _router_body(x_ref, w_ref, b_ref, gates_ref, idx_ref, acc_ref):
    j = pl.program_id(1)
    part = jax.lax.dot_general(
        w_ref[...], x_ref[...], (((1,), (1,)), ((), ())),
        preferred_element_type=jnp.float32,
        precision=jax.lax.Precision.DEFAULT,
    )

    @pl.when(j == 0)
    def _():
        acc_ref[...] = part

    @pl.when(j > 0)
    def _():
        acc_ref[...] = acc_ref[...] + part

    @pl.when(j == ND - 1)
    def _():
        logits = acc_ref[...] + b_ref[...]
        iota_f = jax.lax.broadcasted_iota(jnp.int32, logits.shape, 0).astype(jnp.float32)
        cur = logits
        vals = []
        idxs = []
        for k in range(TOP_K):
            m = jnp.max(cur, axis=0, keepdims=True)
            eq = cur == m
            am = jnp.min(jnp.where(eq, iota_f, 64.0), axis=0, keepdims=True)
            vals.append(m)
            idxs.append(am)
            if k + 1 < TOP_K:
                cur = jnp.where(eq, -jnp.inf, cur)
        topv = jnp.concatenate(vals, axis=0)
        topi = jnp.concatenate(idxs, axis=0)
        e = jnp.exp(topv - topv[:1])
        gates_ref[...] = e / jnp.sum(e, axis=0, keepdims=True)
        idx_ref[...] = topi.astype(jnp.int32)


def kernel(x, W, b):
    B, T, D = x.shape
    n_tok = B * T
    x2 = x.reshape(n_tok, D)
    b2 = b.reshape(N_HEADS, 1)
    grid = (n_tok // BLK, ND)
    gates_t, idx_t = pl.pallas_call(
        _router_body,
        grid=grid,
        in_specs=[
            pl.BlockSpec((BLK, DCHUNK), lambda i, j: (i, j)),
            pl.BlockSpec((N_HEADS, DCHUNK), lambda i, j: (0, j)),
            pl.BlockSpec((N_HEADS, 1), lambda i, j: (0, 0)),
        ],
        out_specs=[
            pl.BlockSpec((TOP_K, BLK), lambda i, j: (0, i)),
            pl.BlockSpec((TOP_K, BLK), lambda i, j: (0, i)),
        ],
        out_shape=[
            jax.ShapeDtypeStruct((TOP_K, n_tok), jnp.float32),
            jax.ShapeDtypeStruct((TOP_K, n_tok), jnp.int32),
        ],
        scratch_shapes=[pltpu.VMEM((N_HEADS, BLK), jnp.float32)],
        compiler_params=pltpu.CompilerParams(
            dimension_semantics=("parallel", "arbitrary")),
    )(x2, W, b2)
    gates = gates_t.T.reshape(B, T, TOP_K)
    idx = idx_t.T.reshape(B, T, TOP_K)
    return gates, idx
